# baseline (speedup 1.0000x reference)
import functools

import jax
import jax.numpy as jnp
from jax import lax
from jax.experimental import pallas as pl
from jax.experimental.pallas import tpu as pltpu
from jax.experimental.pallas import tpu_sc as plsc

_D = 64
_SCALE = float(_D) ** 0.5
_NC, _NS = 2, 16
_NW = _NC * _NS
_B = 4096 * 200
_PER_W = _B // _NW
_CH = 128
_NCHUNK = _PER_W // _CH

_mesh = plsc.VectorSubcoreMesh(core_axis_name="c", subcore_axis_name="s")


@functools.partial(
    pl.kernel,
    out_type=jax.ShapeDtypeStruct((_NW * _NCHUNK, _CH, _D), jnp.float32),
    mesh=_mesh,
    compiler_params=pltpu.CompilerParams(use_tc_tiling_on_sc=False),
    scratch_types=[
        pltpu.VMEM((_NCHUNK, _CH), jnp.int32),
        pltpu.VMEM((_CH, _D), jnp.float32),
        pltpu.SemaphoreType.DMA,
    ],
)
def _emb_lookup(table, idx, out, idx_v, rows_v, sem):
    wid = lax.axis_index("s") * _NC + lax.axis_index("c")
    pltpu.sync_copy(idx.at[wid], idx_v)

    @pl.loop(0, _NCHUNK)
    def _chunk(j):
        pltpu.async_copy(table.at[idx_v.at[j]], rows_v, sem).wait()

        @pl.loop(0, _CH)
        def _row(r):
            for c in range(_D // 16):
                sl = pl.ds(c * 16, 16)
                rows_v[r, sl] = rows_v[r, sl] * _SCALE

        pltpu.sync_copy(rows_v, out.at[wid * _NCHUNK + j])


def kernel(inputs, embeddings):
    idx = inputs.reshape(_NW, _NCHUNK, _CH)
    out = _emb_lookup(embeddings, idx)
    return out.reshape(4096, 200, _D)


# speedup vs baseline: 1.2092x; 1.2092x over previous
import functools

import jax
import jax.numpy as jnp
from jax import lax
from jax.experimental import pallas as pl
from jax.experimental.pallas import tpu as pltpu
from jax.experimental.pallas import tpu_sc as plsc

_D = 64
_SCALE = float(_D) ** 0.5
_NC, _NS = 2, 16
_NW = _NC * _NS
_B = 4096 * 200
_PER_W = _B // _NW
_CH = 128
_NCHUNK = _PER_W // _CH
_NBUF = 4

_mesh = plsc.VectorSubcoreMesh(core_axis_name="c", subcore_axis_name="s")


@functools.partial(
    pl.kernel,
    out_type=jax.ShapeDtypeStruct((_NW * _NCHUNK, _CH, _D), jnp.float32),
    mesh=_mesh,
    compiler_params=pltpu.CompilerParams(use_tc_tiling_on_sc=False),
    scratch_types=[
        pltpu.VMEM((_NCHUNK, _CH), jnp.int32),
        [pltpu.VMEM((_CH, _D), jnp.float32) for _ in range(_NBUF)],
        [pltpu.SemaphoreType.DMA for _ in range(_NBUF)],
        [pltpu.SemaphoreType.DMA for _ in range(_NBUF)],
    ],
)
def _emb_lookup(table, idx, out, idx_v, bufs, gsems, wsems):
    wid = lax.axis_index("s") * _NC + lax.axis_index("c")
    out_base = wid * _NCHUNK
    pltpu.sync_copy(idx.at[wid], idx_v)

    def start_gather(j, b):
        pltpu.async_copy(table.at[idx_v.at[j]], bufs[b], gsems[b])

    def wait_gather(j, b):
        pltpu.make_async_copy(table.at[idx_v.at[j]], bufs[b], gsems[b]).wait()

    def start_wb(j, b):
        pltpu.async_copy(bufs[b], out.at[out_base + j], wsems[b])

    def wait_wb(j, b):
        pltpu.make_async_copy(bufs[b], out.at[out_base + j], wsems[b]).wait()

    def scale(b):
        buf = bufs[b]

        @pl.loop(0, _CH, unroll=8)
        def _row(r):
            for c in range(_D // 16):
                sl = pl.ds(c * 16, 16)
                buf[r, sl] = buf[r, sl] * _SCALE

    def step(j, b, first=False, tail=False):
        wait_gather(j, b)
        scale(b)
        start_wb(j, b)
        if not first:
            wait_wb(j - 1, (b - 1) % _NBUF)
        if not tail:
            start_gather(j + _NBUF - 1, (b - 1) % _NBUF)

    for b in range(_NBUF - 1):
        start_gather(b, b)

    for b in range(_NBUF):
        step(b, b, first=(b == 0))

    @pl.loop(_NBUF, _NCHUNK - _NBUF, step=_NBUF)
    def _block(j0):
        for b in range(_NBUF):
            step(j0 + b, b)

    for b in range(_NBUF):
        j = _NCHUNK - _NBUF + b
        step(j, b, tail=(j + _NBUF - 1 >= _NCHUNK))

    wait_wb(_NCHUNK - 1, (_NCHUNK - 1) % _NBUF)


def kernel(inputs, embeddings):
    idx = inputs.reshape(_NW, _NCHUNK, _CH)
    out = _emb_lookup(embeddings, idx)
    return out.reshape(4096, 200, _D)
